# Initial kernel scaffold; baseline (speedup 1.0000x reference)
import functools

import jax
import jax.numpy as jnp
from jax import lax
from jax.experimental import pallas as pl
from jax.experimental.pallas import tpu as pltpu
from jax.experimental.pallas import tpu_sc as plsc

NC = 2
NS = 16
NW = NC * NS
K = 128
C = 128


def _make_deg(n_pad, nb):
    mesh = plsc.VectorSubcoreMesh(
        core_axis_name="c", subcore_axis_name="s",
        num_cores=NC, num_subcores=NS)
    rows_per_tile = n_pad // NS

    @functools.partial(
        pl.kernel,
        out_type=jax.ShapeDtypeStruct((NC, n_pad, 16), jnp.float32),
        mesh=mesh,
        scratch_types=[
            pltpu.VMEM((nb, K), jnp.int32),
            pltpu.VMEM((K, 16), jnp.float32),
            pltpu.VMEM_SHARED((n_pad, 16), jnp.float32),
        ],
    )
    def deg_kernel(dst_hbm, zeros_hbm, ones_hbm, out_hbm, dst_v, ones_v, acc_sh):
        cid = lax.axis_index("c")
        sid = lax.axis_index("s")
        wid = cid * NS + sid
        pltpu.sync_copy(dst_hbm.at[wid], dst_v)
        pltpu.sync_copy(ones_hbm, ones_v)
        r0 = sid * rows_per_tile
        pltpu.sync_copy(zeros_hbm.at[pl.ds(r0, rows_per_tile)],
                        acc_sh.at[pl.ds(r0, rows_per_tile)])
        plsc.subcore_barrier()

        def body(j, carry):
            pltpu.sync_copy(ones_v, acc_sh.at[dst_v.at[j]], add=True)
            return carry

        lax.fori_loop(0, nb, body, 0)
        plsc.subcore_barrier()
        pltpu.sync_copy(acc_sh.at[pl.ds(r0, rows_per_tile)],
                        out_hbm.at[cid, pl.ds(r0, rows_per_tile)])

    return deg_kernel


def _make_prop(n_pad, nb):
    mesh = plsc.VectorSubcoreMesh(
        core_axis_name="c", subcore_axis_name="s",
        num_cores=NC, num_subcores=NS)
    rows_per_tile = n_pad // NS

    @functools.partial(
        pl.kernel,
        out_type=jax.ShapeDtypeStruct((NC, n_pad, C), jnp.float32),
        mesh=mesh,
        scratch_types=[
            pltpu.VMEM((nb, K), jnp.int32),
            pltpu.VMEM((nb, K), jnp.int32),
            pltpu.VMEM((K, C), jnp.float32),
            pltpu.VMEM_SHARED((n_pad, C), jnp.float32),
            pltpu.SemaphoreType.DMA,
        ],
    )
    def prop_kernel(y_hbm, src_hbm, dst_hbm, zeros_hbm, out_hbm,
                    src_v, dst_v, rows_v, acc_sh, gsem):
        cid = lax.axis_index("c")
        sid = lax.axis_index("s")
        wid = cid * NS + sid
        pltpu.sync_copy(src_hbm.at[wid], src_v)
        pltpu.sync_copy(dst_hbm.at[wid], dst_v)
        r0 = sid * rows_per_tile
        pltpu.sync_copy(zeros_hbm.at[pl.ds(r0, rows_per_tile)],
                        acc_sh.at[pl.ds(r0, rows_per_tile)])
        plsc.subcore_barrier()

        def body(j, carry):
            pltpu.async_copy(y_hbm.at[src_v.at[j]], rows_v, gsem).wait()
            pltpu.sync_copy(rows_v, acc_sh.at[dst_v.at[j]], add=True)
            return carry

        lax.fori_loop(0, nb, body, 0)
        plsc.subcore_barrier()
        pltpu.sync_copy(acc_sh.at[pl.ds(r0, rows_per_tile)],
                        out_hbm.at[cid, pl.ds(r0, rows_per_tile)])

    return prop_kernel


def _tc_first(x_pad, W1, degp, block):
    n_pad = x_pad.shape[0]
    grid = (n_pad // block,)

    def body(x_ref, w_ref, degp_ref, y_ref, xw_ref, dinv_ref):
        dp = degp_ref[0] + degp_ref[1]
        deg = dp[:, 0:1] + 1.0
        dinvb = jnp.broadcast_to(lax.rsqrt(deg), (block, C))
        xw = jnp.dot(x_ref[...], w_ref[...],
                     preferred_element_type=jnp.float32)
        xw_ref[...] = xw
        dinv_ref[...] = dinvb
        y_ref[...] = dinvb * xw

    return pl.pallas_call(
        body,
        grid=grid,
        in_specs=[
            pl.BlockSpec((block, C), lambda i: (i, 0)),
            pl.BlockSpec((C, C), lambda i: (0, 0)),
            pl.BlockSpec((NC, block, 16), lambda i: (0, i, 0)),
        ],
        out_specs=[pl.BlockSpec((block, C), lambda i: (i, 0))] * 3,
        out_shape=[jax.ShapeDtypeStruct((n_pad, C), jnp.float32)] * 3,
    )(x_pad, W1, degp)


def _tc_mid(acc1, xw1, dinvb, b1, W2, block):
    n_pad = xw1.shape[0]
    grid = (n_pad // block,)

    def body(acc_ref, xw_ref, dinv_ref, b_ref, w_ref, y2_ref, xw2_ref):
        dv = dinv_ref[...]
        a = acc_ref[0] + acc_ref[1]
        h = jnp.maximum(dv * a + dv * dv * xw_ref[...] + b_ref[...], 0.0)
        xw2 = jnp.dot(h, w_ref[...], preferred_element_type=jnp.float32)
        xw2_ref[...] = xw2
        y2_ref[...] = dv * xw2

    return pl.pallas_call(
        body,
        grid=grid,
        in_specs=[
            pl.BlockSpec((NC, block, C), lambda i: (0, i, 0)),
            pl.BlockSpec((block, C), lambda i: (i, 0)),
            pl.BlockSpec((block, C), lambda i: (i, 0)),
            pl.BlockSpec((1, C), lambda i: (0, 0)),
            pl.BlockSpec((C, C), lambda i: (0, 0)),
        ],
        out_specs=[pl.BlockSpec((block, C), lambda i: (i, 0))] * 2,
        out_shape=[jax.ShapeDtypeStruct((n_pad, C), jnp.float32)] * 2,
    )(acc1, xw1, dinvb, b1, W2)


def _tc_last(acc2, xw2, dinvb, b2, block):
    n_pad = xw2.shape[0]
    grid = (n_pad // block,)

    def body(acc_ref, xw_ref, dinv_ref, b_ref, o_ref):
        dv = dinv_ref[...]
        a = acc_ref[0] + acc_ref[1]
        o_ref[...] = dv * a + dv * dv * xw_ref[...] + b_ref[...]

    return pl.pallas_call(
        body,
        grid=grid,
        in_specs=[
            pl.BlockSpec((NC, block, C), lambda i: (0, i, 0)),
            pl.BlockSpec((block, C), lambda i: (i, 0)),
            pl.BlockSpec((block, C), lambda i: (i, 0)),
            pl.BlockSpec((1, C), lambda i: (0, 0)),
        ],
        out_specs=pl.BlockSpec((block, C), lambda i: (i, 0)),
        out_shape=jax.ShapeDtypeStruct((n_pad, C), jnp.float32),
    )(acc2, xw2, dinvb, b2)


def kernel(x, edge_index, W1, b1, W2, b2):
    n, c = x.shape
    e = edge_index.shape[1]
    assert c == C

    nb = -(-e // (NW * K))
    if nb % 2:
        nb += 1
    e_pad = NW * K * nb
    n_pad = 10112 if n == 10000 else ((n + 1 + 127) // 128) * 128
    block = n_pad // 8

    si = edge_index[0].astype(jnp.int32)
    di = edge_index[1].astype(jnp.int32)
    pad = jnp.full((e_pad - e,), n, jnp.int32)
    si = jnp.concatenate([si, pad]).reshape(NW, nb, K)
    di = jnp.concatenate([di, pad]).reshape(NW, nb, K)
    x_pad = jnp.zeros((n_pad, C), jnp.float32).at[:n].set(x)

    zeros128 = jnp.zeros((n_pad, C), jnp.float32)
    zeros16 = jnp.zeros((n_pad, 16), jnp.float32)
    ones16 = jnp.ones((K, 16), jnp.float32)

    deg_fn = _make_deg(n_pad, nb)
    prop_fn = _make_prop(n_pad, nb)

    degp = deg_fn(di, zeros16, ones16)
    y1, xw1, dinvb = _tc_first(x_pad, W1, degp, block)
    acc1 = prop_fn(y1, si, di, zeros128)
    y2, xw2 = _tc_mid(acc1, xw1, dinvb, b1.reshape(1, C), W2, block)
    acc2 = prop_fn(y2, si, di, zeros128)
    out = _tc_last(acc2, xw2, dinvb, b2.reshape(1, C), block)
    return out[:n]


# speedup vs baseline: 6.8914x; 6.8914x over previous
import functools

import jax
import jax.numpy as jnp
import numpy as np
from jax import lax
from jax.experimental import pallas as pl
from jax.experimental.pallas import tpu as pltpu
from jax.experimental.pallas import tpu_sc as plsc

NC = 2
NS = 16
NW = NC * NS
K = 128
C = 128
_Z = np.int32(0)

_MESH = dict(core_axis_name="c", subcore_axis_name="s",
             num_cores=NC, num_subcores=NS)


def _make_prop(n_pad, nb):
    mesh = plsc.VectorSubcoreMesh(**_MESH)
    rows_per_tile = n_pad // NS

    @functools.partial(
        pl.kernel,
        out_type=jax.ShapeDtypeStruct((NC, n_pad, C), jnp.float32),
        mesh=mesh,
        scratch_types=[
            pltpu.VMEM((nb, K), jnp.int32),
            pltpu.VMEM((4, K), jnp.int32),
            pltpu.VMEM((K, C), jnp.float32),
            pltpu.VMEM((K, C), jnp.float32),
            pltpu.VMEM_SHARED((n_pad, C), jnp.float32),
        ] + [pltpu.SemaphoreType.DMA] * 4,
    )
    def prop_kernel(y_hbm, src_hbm, dst_hbm, out_hbm,
                    src_v, dst_c, b0, b1, acc_sh, g0, g1, s0, s1):
        cid = lax.axis_index("c")
        sid = lax.axis_index("s")
        wid = cid * NS + sid
        pltpu.sync_copy(src_hbm.at[wid], src_v)
        r0 = sid * rows_per_tile

        zv = jnp.zeros((16,), jnp.float32)

        def zfill(i, carry):
            for c8 in range(C // 16):
                b0[i, pl.ds(c8 * 16, 16)] = zv
            return carry

        lax.fori_loop(jnp.int32(0), jnp.int32(K), zfill, jnp.int32(0))
        for r in range(rows_per_tile // K):
            pltpu.sync_copy(b0, acc_sh.at[pl.ds(r0 + r * K, K)])
        plsc.subcore_barrier()

        def body(G, carry):
            pltpu.sync_copy(dst_hbm.at[wid, pl.ds(G * 4, 4)], dst_c)
            for p in range(2):
                j0 = G * 4 + 2 * p
                j1 = j0 + 1
                d0 = jnp.int32(2 * p)
                d1 = jnp.int32(2 * p + 1)
                pltpu.async_copy(y_hbm.at[src_v.at[j0]], b0, g0)
                pltpu.async_copy(y_hbm.at[src_v.at[j1]], b1, g1)
                pltpu.make_async_copy(y_hbm.at[src_v.at[j0]], b0, g0).wait()
                pltpu.async_copy(b0, acc_sh.at[dst_c.at[d0]], s0, add=True)
                pltpu.make_async_copy(y_hbm.at[src_v.at[j1]], b1, g1).wait()
                pltpu.async_copy(b1, acc_sh.at[dst_c.at[d1]], s1, add=True)
                pltpu.make_async_copy(b0, acc_sh.at[dst_c.at[d0]], s0).wait()
                pltpu.make_async_copy(b1, acc_sh.at[dst_c.at[d1]], s1).wait()
            return carry

        lax.fori_loop(jnp.int32(0), jnp.int32(nb // 4), body, jnp.int32(0))
        plsc.subcore_barrier()
        pltpu.sync_copy(acc_sh.at[pl.ds(r0, rows_per_tile)],
                        out_hbm.at[cid, pl.ds(r0, rows_per_tile)])

    return prop_kernel


def _tc_first(x_pad, W1, degp, block):
    n_pad = x_pad.shape[0]
    grid = (n_pad // block,)

    def body(x_ref, w_ref, degp_ref, y_ref, xw_ref, dinv_ref):
        dp = degp_ref[0] + degp_ref[1]
        deg = dp[:, 0:1] + 1.0
        dinvb = jnp.broadcast_to(lax.rsqrt(deg), (block, C))
        xw = jnp.dot(x_ref[...], w_ref[...],
                     preferred_element_type=jnp.float32)
        xw_ref[...] = xw
        dinv_ref[...] = dinvb
        y_ref[...] = dinvb * xw

    return pl.pallas_call(
        body,
        grid=grid,
        in_specs=[
            pl.BlockSpec((block, C), lambda i: (i, _Z)),
            pl.BlockSpec((C, C), lambda i: (_Z, _Z)),
            pl.BlockSpec((NC, block, C), lambda i: (_Z, i, _Z)),
        ],
        out_specs=[pl.BlockSpec((block, C), lambda i: (i, _Z))] * 3,
        out_shape=[jax.ShapeDtypeStruct((n_pad, C), jnp.float32)] * 3,
    )(x_pad, W1, degp)


def _tc_mid(acc1, xw1, dinvb, b1, W2, block):
    n_pad = xw1.shape[0]
    grid = (n_pad // block,)

    def body(acc_ref, xw_ref, dinv_ref, b_ref, w_ref, y2_ref, xw2_ref):
        dv = dinv_ref[...]
        a = acc_ref[0] + acc_ref[1]
        h = jnp.maximum(dv * a + dv * dv * xw_ref[...] + b_ref[...], 0.0)
        xw2 = jnp.dot(h, w_ref[...], preferred_element_type=jnp.float32)
        xw2_ref[...] = xw2
        y2_ref[...] = dv * xw2

    return pl.pallas_call(
        body,
        grid=grid,
        in_specs=[
            pl.BlockSpec((NC, block, C), lambda i: (_Z, i, _Z)),
            pl.BlockSpec((block, C), lambda i: (i, _Z)),
            pl.BlockSpec((block, C), lambda i: (i, _Z)),
            pl.BlockSpec((1, C), lambda i: (_Z, _Z)),
            pl.BlockSpec((C, C), lambda i: (_Z, _Z)),
        ],
        out_specs=[pl.BlockSpec((block, C), lambda i: (i, _Z))] * 2,
        out_shape=[jax.ShapeDtypeStruct((n_pad, C), jnp.float32)] * 2,
    )(acc1, xw1, dinvb, b1, W2)


def _tc_last(acc2, xw2, dinvb, b2, block):
    n_pad = xw2.shape[0]
    grid = (n_pad // block,)

    def body(acc_ref, xw_ref, dinv_ref, b_ref, o_ref):
        dv = dinv_ref[...]
        a = acc_ref[0] + acc_ref[1]
        o_ref[...] = dv * a + dv * dv * xw_ref[...] + b_ref[...]

    return pl.pallas_call(
        body,
        grid=grid,
        in_specs=[
            pl.BlockSpec((NC, block, C), lambda i: (_Z, i, _Z)),
            pl.BlockSpec((block, C), lambda i: (i, _Z)),
            pl.BlockSpec((block, C), lambda i: (i, _Z)),
            pl.BlockSpec((1, C), lambda i: (_Z, _Z)),
        ],
        out_specs=pl.BlockSpec((block, C), lambda i: (i, _Z)),
        out_shape=jax.ShapeDtypeStruct((n_pad, C), jnp.float32),
    )(acc2, xw2, dinvb, b2)


def kernel(x, edge_index, W1, b1, W2, b2):
    n, c = x.shape
    e = edge_index.shape[1]
    assert c == C

    nb = -(-e // (NW * K))
    nb = ((nb + 3) // 4) * 4
    e_pad = NW * K * nb
    n_pad = ((n + 1 + NS * K - 1) // (NS * K)) * (NS * K)
    block = n_pad // 8

    si = edge_index[0].astype(jnp.int32)
    di = edge_index[1].astype(jnp.int32)
    pad = jnp.full((e_pad - e,), n, jnp.int32)
    si = jnp.concatenate([si, pad]).reshape(NW, nb, K)
    di = jnp.concatenate([di, pad]).reshape(NW, nb, K)
    x_pad = jnp.zeros((n_pad, C), jnp.float32).at[:n].set(x)

    prop_fn = _make_prop(n_pad, nb)

    ones128 = jnp.ones((n_pad, C), jnp.float32)
    degp = prop_fn(ones128, si, di)
    y1, xw1, dinvb = _tc_first(x_pad, W1, degp, block)
    acc1 = prop_fn(y1, si, di)
    y2, xw2 = _tc_mid(acc1, xw1, dinvb, b1.reshape(1, C), W2, block)
    acc2 = prop_fn(y2, si, di)
    out = _tc_last(acc2, xw2, dinvb, b2.reshape(1, C), block)
    return out[:n]
